# baseline (speedup 1.0000x reference)
import functools

import jax
import jax.numpy as jnp
from jax import lax
from jax.experimental import pallas as pl
from jax.experimental.pallas import tpu as pltpu
from jax.experimental.pallas import tpu_sc as plsc

VOCAB = 40960
D = 128
B = 16384
NF = 30

NUM_CORES = 2
NUM_SUBCORES = 16
LANES = 16
NW = NUM_CORES * NUM_SUBCORES
S_PER_W = B // NW
D_VECS = D // LANES


def _sc_body(idx_hbm, table_hbm, out_hbm, *refs):
    ibufs = refs[:NF]
    accb, isem, sem = refs[NF], refs[NF + 1], refs[NF + 2]
    wid = lax.axis_index("s") * NUM_CORES + lax.axis_index("c")
    sbase = wid * S_PER_W

    for j in range(NF):
        pltpu.async_copy(
            idx_hbm.at[pl.ds(j * B + sbase, S_PER_W)], ibufs[j], isem)

    zero = jnp.zeros((LANES,), jnp.float32)

    def zero_row(r, _):
        for d in range(D_VECS):
            accb[r, pl.ds(d * LANES, LANES)] = zero
        return 0

    lax.fori_loop(0, S_PER_W, zero_row, 0)

    for j in range(NF):
        pltpu.make_async_copy(
            idx_hbm.at[pl.ds(j * B + sbase, S_PER_W)], ibufs[j], isem).wait()
        pltpu.async_copy(table_hbm.at[ibufs[j]], accb, sem, add=True)
    for j in range(NF):
        pltpu.make_async_copy(table_hbm.at[ibufs[j]], accb, sem).wait()

    pltpu.sync_copy(accb, out_hbm.at[pl.ds(sbase, S_PER_W)])


_sc_gather_sum = functools.partial(
    pl.kernel,
    out_type=jax.ShapeDtypeStruct((B, D), jnp.float32),
    mesh=plsc.VectorSubcoreMesh(
        core_axis_name="c", subcore_axis_name="s",
        num_cores=NUM_CORES, num_subcores=NUM_SUBCORES),
    scratch_types=(
        [pltpu.VMEM((S_PER_W,), jnp.int32) for _ in range(NF)]
        + [pltpu.VMEM((S_PER_W, D), jnp.float32),
           pltpu.SemaphoreType.DMA,
           pltpu.SemaphoreType.DMA]
    ),
)(_sc_body)


def _mlp_body(x_ref, w1_ref, b1_ref, w2_ref, b2_ref, w3_ref, b3_ref, o_ref):
    x = jnp.clip(x_ref[...], 0.0, 1.0)
    h = jnp.dot(x, w1_ref[...], preferred_element_type=jnp.float32)
    h = jnp.clip(h + b1_ref[...], 0.0, 1.0)
    h = jnp.dot(h, w2_ref[...], preferred_element_type=jnp.float32)
    h = jnp.clip(h + b2_ref[...], 0.0, 1.0)
    o_ref[...] = (jnp.dot(h, w3_ref[...], preferred_element_type=jnp.float32)
                  + b3_ref[...])


def kernel(idxs, table, w1, b1, w2, b2, w3, b3):
    idx_t = idxs.T.reshape(NF * B)
    acc = _sc_gather_sum(idx_t, table)
    bk = 2048
    def wspec(shape):
        return pl.BlockSpec(shape, lambda i: (0, 0))
    out = pl.pallas_call(
        _mlp_body,
        grid=(B // bk,),
        in_specs=[pl.BlockSpec((bk, D), lambda i: (i, 0)),
                  wspec((D, 32)), wspec((1, 32)), wspec((32, 32)),
                  wspec((1, 32)), wspec((32, 1)), wspec((1, 1))],
        out_specs=pl.BlockSpec((bk, 1), lambda i: (i, 0)),
        out_shape=jax.ShapeDtypeStruct((B, 1), jnp.float32),
    )(acc, w1, b1.reshape(1, 32), w2, b2.reshape(1, 32),
      w3, b3.reshape(1, 1))
    return out


# speedup vs baseline: 1.0348x; 1.0348x over previous
import functools

import jax
import jax.numpy as jnp
from jax import lax
from jax.experimental import pallas as pl
from jax.experimental.pallas import tpu as pltpu
from jax.experimental.pallas import tpu_sc as plsc

VOCAB = 40960
D = 128
B = 16384
NF = 30

NUM_CORES = 2
NUM_SUBCORES = 16
LANES = 16
NW = NUM_CORES * NUM_SUBCORES
S_PER_W = B // NW
D_VECS = D // LANES


def _sc_body(idx_hbm, table_hbm, out_hbm, *refs):
    ibufs = refs[:NF]
    accb, isem, sem = refs[NF], refs[NF + 1], refs[NF + 2]
    wid = lax.axis_index("s") * NUM_CORES + lax.axis_index("c")
    sbase = wid * S_PER_W

    for j in range(NF):
        pltpu.async_copy(
            idx_hbm.at[pl.ds(j * B + sbase, S_PER_W)], ibufs[j], isem)

    zero = jnp.zeros((LANES,), jnp.float32)

    def zero_row(r, _):
        for d in range(D_VECS):
            accb[r, pl.ds(d * LANES, LANES)] = zero
        return 0

    lax.fori_loop(0, S_PER_W, zero_row, 0)

    for j in range(NF):
        pltpu.make_async_copy(
            idx_hbm.at[pl.ds(j * B + sbase, S_PER_W)], ibufs[j], isem).wait()
        pltpu.async_copy(table_hbm.at[ibufs[j]], accb, sem, add=True)
    for j in range(NF):
        pltpu.make_async_copy(table_hbm.at[ibufs[j]], accb, sem).wait()

    pltpu.sync_copy(accb, out_hbm.at[pl.ds(sbase, S_PER_W)])


_sc_gather_sum = functools.partial(
    pl.kernel,
    out_type=jax.ShapeDtypeStruct((B, D), jnp.float32),
    mesh=plsc.VectorSubcoreMesh(
        core_axis_name="c", subcore_axis_name="s",
        num_cores=NUM_CORES, num_subcores=NUM_SUBCORES),
    scratch_types=(
        [pltpu.VMEM((S_PER_W,), jnp.int32) for _ in range(NF)]
        + [pltpu.VMEM((S_PER_W, D), jnp.float32),
           pltpu.SemaphoreType.DMA,
           pltpu.SemaphoreType.DMA]
    ),
)(_sc_body)


def _mlp_body(x_ref, w1_ref, b1_ref, w2_ref, b2_ref, w3_ref, b3_ref, o_ref):
    x = jnp.clip(x_ref[...], 0.0, 1.0)
    h = jnp.dot(x, w1_ref[...], preferred_element_type=jnp.float32)
    h = jnp.clip(h + b1_ref[...], 0.0, 1.0)
    h = jnp.dot(h, w2_ref[...], preferred_element_type=jnp.float32)
    h = jnp.clip(h + b2_ref[...], 0.0, 1.0)
    o_ref[...] = (jnp.dot(h, w3_ref[...], preferred_element_type=jnp.float32)
                  + b3_ref[...])[:, 0]


def kernel(idxs, table, w1, b1, w2, b2, w3, b3):
    idx_t = idxs.T.reshape(NF * B)
    acc = _sc_gather_sum(idx_t, table)
    bk = 4096
    def wspec(shape):
        return pl.BlockSpec(shape, lambda i: (0, 0))
    out = pl.pallas_call(
        _mlp_body,
        grid=(B // bk,),
        in_specs=[pl.BlockSpec((bk, D), lambda i: (i, 0)),
                  wspec((D, 32)), wspec((1, 32)), wspec((32, 32)),
                  wspec((1, 32)), wspec((32, 1)), wspec((1, 1))],
        out_specs=pl.BlockSpec((bk,), lambda i: (i,)),
        out_shape=jax.ShapeDtypeStruct((B,), jnp.float32),
    )(acc, w1, b1.reshape(1, 32), w2, b2.reshape(1, 32),
      w3, b3.reshape(1, 1))
    return out.reshape(B, 1)


# speedup vs baseline: 1.0366x; 1.0018x over previous
import functools

import jax
import jax.numpy as jnp
from jax import lax
from jax.experimental import pallas as pl
from jax.experimental.pallas import tpu as pltpu
from jax.experimental.pallas import tpu_sc as plsc

VOCAB = 40960
D = 128
B = 16384
NF = 30

NUM_CORES = 2
NUM_SUBCORES = 16
LANES = 16
NW = NUM_CORES * NUM_SUBCORES
S_PER_W = B // NW
D_VECS = D // LANES


def _sc_body(idx_hbm, table_hbm, out_hbm, *refs):
    ibufs = refs[:NF]
    accb, isem, sem = refs[NF], refs[NF + 1], refs[NF + 2]
    wid = lax.axis_index("s") * NUM_CORES + lax.axis_index("c")
    sbase = wid * S_PER_W

    for j in range(NF):
        pltpu.async_copy(
            idx_hbm.at[pl.ds(j * B + sbase, S_PER_W)], ibufs[j], isem)

    zero = jnp.zeros((LANES,), jnp.float32)

    def zero_row(r, _):
        for d in range(D_VECS):
            accb[r, pl.ds(d * LANES, LANES)] = zero
        return 0

    lax.fori_loop(0, S_PER_W, zero_row, 0)

    for j in range(NF):
        pltpu.make_async_copy(
            idx_hbm.at[pl.ds(j * B + sbase, S_PER_W)], ibufs[j], isem).wait()
        pltpu.async_copy(table_hbm.at[ibufs[j]], accb, sem, add=True)
    for j in range(NF):
        pltpu.make_async_copy(table_hbm.at[ibufs[j]], accb, sem).wait()

    pltpu.sync_copy(accb, out_hbm.at[pl.ds(sbase, S_PER_W)])


_sc_gather_sum = functools.partial(
    pl.kernel,
    out_type=jax.ShapeDtypeStruct((B, D), jnp.float32),
    mesh=plsc.VectorSubcoreMesh(
        core_axis_name="c", subcore_axis_name="s",
        num_cores=NUM_CORES, num_subcores=NUM_SUBCORES),
    scratch_types=(
        [pltpu.VMEM((S_PER_W,), jnp.int32) for _ in range(NF)]
        + [pltpu.VMEM((S_PER_W, D), jnp.float32),
           pltpu.SemaphoreType.DMA,
           pltpu.SemaphoreType.DMA]
    ),
)(_sc_body)


def _mlp_body(x_ref, w1_ref, b1_ref, w2_ref, b2_ref, w3_ref, b3_ref, o_ref):
    x = jnp.clip(x_ref[...], 0.0, 1.0)
    h = jnp.dot(x, w1_ref[...], preferred_element_type=jnp.float32)
    h = jnp.clip(h + b1_ref[...], 0.0, 1.0)
    h = jnp.dot(h, w2_ref[...], preferred_element_type=jnp.float32)
    h = jnp.clip(h + b2_ref[...], 0.0, 1.0)
    o_ref[...] = (jnp.dot(h, w3_ref[...], preferred_element_type=jnp.float32)
                  + b3_ref[...])[:, 0]


def kernel(idxs, table, w1, b1, w2, b2, w3, b3):
    idx_t = idxs.T.reshape(NF * B)
    acc = _sc_gather_sum(idx_t, table)
    bk = 8192
    def wspec(shape):
        return pl.BlockSpec(shape, lambda i: (0, 0))
    out = pl.pallas_call(
        _mlp_body,
        grid=(B // bk,),
        in_specs=[pl.BlockSpec((bk, D), lambda i: (i, 0)),
                  wspec((D, 32)), wspec((1, 32)), wspec((32, 32)),
                  wspec((1, 32)), wspec((32, 1)), wspec((1, 1))],
        out_specs=pl.BlockSpec((bk,), lambda i: (i,)),
        out_shape=jax.ShapeDtypeStruct((B,), jnp.float32),
    )(acc, w1, b1.reshape(1, 32), w2, b2.reshape(1, 32),
      w3, b3.reshape(1, 1))
    return out.reshape(B, 1)
